# baseline (speedup 1.0000x reference)
import functools

import jax
import jax.numpy as jnp
from jax import lax
from jax.experimental import pallas as pl
from jax.experimental.pallas import tpu as pltpu
from jax.experimental.pallas import tpu_sc as plsc

EMBED_DIM = 128
NUM_CORES = 2
NUM_SUBCORES = 16
NW = NUM_CORES * NUM_SUBCORES
CHUNK = 128
NB = 3


@functools.lru_cache(maxsize=None)
def _make_gather(total, dim):
    per_w = total // NW
    n_chunks = per_w // CHUNK
    assert n_chunks >= 6 and (n_chunks - 5) % NB == 0
    mesh = plsc.VectorSubcoreMesh(core_axis_name="c", subcore_axis_name="s")

    @functools.partial(
        pl.kernel,
        out_type=jax.ShapeDtypeStruct((total, dim), jnp.float32),
        mesh=mesh,
        scratch_types=[
            pltpu.VMEM((n_chunks, CHUNK), jnp.int32),
            pltpu.VMEM((NB, CHUNK, dim), jnp.float32),
            pltpu.SemaphoreType.DMA((NB,)),
            pltpu.SemaphoreType.DMA((NB,)),
        ],
    )
    def gather_kernel(idx_hbm, table_hbm, out_hbm, idx_v, rows_v, sem_g, sem_s):
        wid = lax.axis_index("s") * NUM_CORES + lax.axis_index("c")
        base = wid * per_w

        pltpu.sync_copy(idx_hbm.at[wid], idx_v)

        def gather(c, i):
            pltpu.async_copy(table_hbm.at[idx_v.at[c]], rows_v.at[i], sem_g.at[i])

        def store(c, i):
            pltpu.async_copy(rows_v.at[i], out_hbm.at[pl.ds(base + c * CHUNK, CHUNK)],
                             sem_s.at[i])

        def wait_gather(i):
            pltpu.make_async_copy(table_hbm.at[idx_v.at[0]], rows_v.at[i],
                                  sem_g.at[i]).wait()

        def wait_store(i):
            pltpu.make_async_copy(rows_v.at[i], out_hbm.at[pl.ds(base, CHUNK)],
                                  sem_s.at[i]).wait()

        def step(t, i):
            wait_gather(i)
            store(t, i)
            wait_store((i + 2) % NB)
            gather(t + 2, (i + 2) % NB)

        gather(0, 0)
        gather(1, 1)
        wait_gather(0)
        store(0, 0)
        gather(2, 2)
        step(1, 1)
        step(2, 2)

        def body(j, carry):
            t = 3 + NB * j
            step(t, 0)
            step(t + 1, 1)
            step(t + 2, 2)
            return carry

        lax.fori_loop(0, (n_chunks - 5) // NB, body, 0)

        t = n_chunks - 2
        wait_gather(t % NB)
        store(t, t % NB)
        wait_store((t + 2) % NB)
        t = n_chunks - 1
        wait_gather(t % NB)
        store(t, t % NB)
        wait_store((t + 2) % NB)
        wait_store(t % NB)

    return gather_kernel


def kernel(indices, table):
    batch, tokens = indices.shape
    total = batch * tokens
    per_w = total // NW
    idx3 = indices.reshape(NW, per_w // CHUNK, CHUNK).astype(jnp.int32)
    out = _make_gather(total, table.shape[1])(idx3, table)
    return out.reshape(batch, tokens, table.shape[1])


# speedup vs baseline: 1.0004x; 1.0004x over previous
import functools

import jax
import jax.numpy as jnp
from jax import lax
from jax.experimental import pallas as pl
from jax.experimental.pallas import tpu as pltpu
from jax.experimental.pallas import tpu_sc as plsc

EMBED_DIM = 128
NUM_CORES = 2
NUM_SUBCORES = 16
NW = NUM_CORES * NUM_SUBCORES
CHUNK = 128
K = 2
SB = K * CHUNK
NB = 3


@functools.lru_cache(maxsize=None)
def _make_gather(total, dim):
    per_w = total // NW
    n_chunks = per_w // SB
    assert n_chunks >= 6
    mesh = plsc.VectorSubcoreMesh(core_axis_name="c", subcore_axis_name="s")

    @functools.partial(
        pl.kernel,
        out_type=jax.ShapeDtypeStruct((total, dim), jnp.float32),
        mesh=mesh,
        scratch_types=[
            pltpu.VMEM((n_chunks * K, CHUNK), jnp.int32),
            pltpu.VMEM((NB, SB, dim), jnp.float32),
            pltpu.SemaphoreType.DMA((NB,)),
            pltpu.SemaphoreType.DMA((NB,)),
        ],
    )
    def gather_kernel(idx_hbm, table_hbm, out_hbm, idx_v, rows_v, sem_g, sem_s):
        wid = lax.axis_index("s") * NUM_CORES + lax.axis_index("c")
        base = wid * per_w

        pltpu.sync_copy(idx_hbm.at[wid], idx_v)

        def gather(c, i):
            for k in range(K):
                pltpu.async_copy(table_hbm.at[idx_v.at[c * K + k]],
                                 rows_v.at[i].at[pl.ds(k * CHUNK, CHUNK)],
                                 sem_g.at[i])

        def store(c, i):
            pltpu.async_copy(rows_v.at[i], out_hbm.at[pl.ds(base + c * SB, SB)],
                             sem_s.at[i])

        def wait_gather(i):
            for k in range(K):
                pltpu.make_async_copy(table_hbm.at[idx_v.at[0]],
                                      rows_v.at[i].at[pl.ds(0, CHUNK)],
                                      sem_g.at[i]).wait()

        def wait_store(i):
            pltpu.make_async_copy(rows_v.at[i], out_hbm.at[pl.ds(base, SB)],
                                  sem_s.at[i]).wait()

        def step(t, i):
            wait_gather(i)
            store(t, i)
            wait_store((i + 2) % NB)
            gather(t + 2, (i + 2) % NB)

        gather(0, 0)
        gather(1, 1)
        wait_gather(0)
        store(0, 0)
        gather(2, 2)
        step(1, 1)
        step(2, 2)

        n_steps = n_chunks - 5
        n_blocks = n_steps // NB

        def body(j, carry):
            t = 3 + NB * j
            step(t, 0)
            step(t + 1, 1)
            step(t + 2, 2)
            return carry

        lax.fori_loop(0, n_blocks, body, 0)

        for t in range(3 + NB * n_blocks, n_chunks - 2):
            step(t, t % NB)

        t = n_chunks - 2
        wait_gather(t % NB)
        store(t, t % NB)
        wait_store((t + 2) % NB)
        t = n_chunks - 1
        wait_gather(t % NB)
        store(t, t % NB)
        wait_store((t + 2) % NB)
        wait_store(t % NB)

    return gather_kernel


def kernel(indices, table):
    batch, tokens = indices.shape
    total = batch * tokens
    per_w = total // NW
    idx3 = indices.reshape(NW, per_w // CHUNK, CHUNK).astype(jnp.int32)
    out = _make_gather(total, table.shape[1])(idx3, table)
    return out.reshape(batch, tokens, table.shape[1])


# speedup vs baseline: 1.0545x; 1.0541x over previous
import functools

import jax
import jax.numpy as jnp
from jax import lax
from jax.experimental import pallas as pl
from jax.experimental.pallas import tpu as pltpu
from jax.experimental.pallas import tpu_sc as plsc

EMBED_DIM = 128
NUM_CORES = 2
NUM_SUBCORES = 16
NW = NUM_CORES * NUM_SUBCORES
CHUNK = 128
NB = 3


@functools.lru_cache(maxsize=None)
def _make_gather(total, dim):
    per_w = total // NW
    n_chunks = per_w // CHUNK
    assert n_chunks >= 8
    mesh = plsc.VectorSubcoreMesh(core_axis_name="c", subcore_axis_name="s")

    @functools.partial(
        pl.kernel,
        out_type=jax.ShapeDtypeStruct((total, dim), jnp.float32),
        mesh=mesh,
        scratch_types=[
            pltpu.VMEM((n_chunks, CHUNK), jnp.int32),
            pltpu.VMEM((NB, CHUNK, dim), jnp.float32),
            pltpu.VMEM_SHARED((NUM_SUBCORES, NB, CHUNK, dim), jnp.float32),
            pltpu.SemaphoreType.DMA((NB,)),
            pltpu.SemaphoreType.DMA((NB,)),
            pltpu.SemaphoreType.DMA((NB,)),
        ],
    )
    def gather_kernel(idx_hbm, table_hbm, out_hbm, idx_v, rows_v, sp_v,
                      sem_g, sem_d, sem_s):
        sid = lax.axis_index("s")
        wid = sid * NUM_CORES + lax.axis_index("c")
        base = wid * per_w

        pltpu.sync_copy(idx_hbm.at[wid], idx_v)

        def gather(c, i):
            pltpu.async_copy(table_hbm.at[idx_v.at[c]], rows_v.at[i],
                             sem_g.at[i])

        def wait_gather(i):
            pltpu.make_async_copy(table_hbm.at[idx_v.at[0]], rows_v.at[i],
                                  sem_g.at[i]).wait()

        def dma(i):
            pltpu.async_copy(rows_v.at[i], sp_v.at[sid, i], sem_d.at[i])

        def wait_dma(i):
            pltpu.make_async_copy(rows_v.at[i], sp_v.at[sid, i],
                                  sem_d.at[i]).wait()

        def store(c, i):
            pltpu.async_copy(sp_v.at[sid, i],
                             out_hbm.at[pl.ds(base + c * CHUNK, CHUNK)],
                             sem_s.at[i])

        def wait_store(i):
            pltpu.make_async_copy(sp_v.at[sid, i], out_hbm.at[pl.ds(base, CHUNK)],
                                  sem_s.at[i]).wait()

        def step(t, i):
            wait_gather(i)
            wait_store(i)
            dma(i)
            wait_dma((i + 2) % NB)
            store(t - 1, (i + 2) % NB)
            gather(t + 2, (i + 2) % NB)

        gather(0, 0)
        gather(1, 1)
        wait_gather(0)
        dma(0)
        gather(2, 2)
        wait_gather(1)
        dma(1)
        wait_dma(0)
        store(0, 0)
        gather(3, 0)
        wait_gather(2)
        dma(2)
        wait_dma(1)
        store(1, 1)
        gather(4, 1)

        n_steps = n_chunks - 5
        n_blocks = n_steps // NB

        def body(j, carry):
            t = 3 + NB * j
            step(t, 0)
            step(t + 1, 1)
            step(t + 2, 2)
            return carry

        lax.fori_loop(0, n_blocks, body, 0)

        for t in range(3 + NB * n_blocks, n_chunks - 2):
            step(t, t % NB)

        for t in (n_chunks - 2, n_chunks - 1):
            i = t % NB
            wait_gather(i)
            wait_store(i)
            dma(i)
            wait_dma((i + 2) % NB)
            store(t - 1, (i + 2) % NB)
        i = (n_chunks - 1) % NB
        wait_dma(i)
        store(n_chunks - 1, i)
        wait_store((i + 1) % NB)
        wait_store((i + 2) % NB)
        wait_store(i)

    return gather_kernel


def kernel(indices, table):
    batch, tokens = indices.shape
    total = batch * tokens
    per_w = total // NW
    idx3 = indices.reshape(NW, per_w // CHUNK, CHUNK).astype(jnp.int32)
    out = _make_gather(total, table.shape[1])(idx3, table)
    return out.reshape(batch, tokens, table.shape[1])
